# Initial kernel scaffold; baseline (speedup 1.0000x reference)
import jax
import jax.numpy as jnp
from jax.experimental import pallas as pl


def kernel(x, edge_index, edge_attr, entity_emb, relation_emb, relation_prior, W_msg1, W_self1, W_agg1, W_msg2, W_self2, W_agg2):
    raise NotImplementedError("write your pallas kernel here")


# speedup vs baseline: 4.8219x; 4.8219x over previous
import functools

import jax
import jax.numpy as jnp
from jax import lax
from jax.experimental import pallas as pl
from jax.experimental.pallas import tpu as pltpu
from jax.experimental.pallas import tpu_sc as plsc

N_NODES = 10000
D = 128
REL_PAD = 512
NC = 2
NS = 16
NW = NC * NS


def _vmesh():
    return plsc.VectorSubcoreMesh(core_axis_name="c", subcore_axis_name="s")


def _sc_gather(table, idx):
    b = idx.shape[0]
    ch = 40
    n_ch = b // ch
    n_loop = (n_ch + NW - 1) // NW

    @functools.partial(
        pl.kernel,
        out_type=jax.ShapeDtypeStruct((b, D), jnp.float32),
        mesh=_vmesh(),
        scratch_types=[
            pltpu.VMEM((ch,), jnp.int32),
            pltpu.VMEM((ch, D), jnp.float32),
            pltpu.SemaphoreType.DMA,
        ],
    )
    def k(table_hbm, idx_hbm, out_hbm, idx_v, rows_v, sem):
        w = lax.axis_index("c") * NS + lax.axis_index("s")

        @pl.loop(0, n_loop)
        def _(j):
            cidx = j * NW + w

            @pl.when(cidx < n_ch)
            def _():
                base = cidx * ch
                pltpu.sync_copy(idx_hbm.at[pl.ds(base, ch)], idx_v)
                pltpu.async_copy(table_hbm.at[idx_v], rows_v, sem).wait()
                pltpu.sync_copy(rows_v, out_hbm.at[pl.ds(base, ch)])

    return k(table, idx)


def _sc_edge(hw, ew, pb, src, dst, attr):
    e_total = src.shape[0]
    ept = e_total // NW
    c_sz = 80
    n_chunk = ept // c_sz
    z_ch = 80
    n_zch = N_NODES // z_ch
    n_zloop = (n_zch + NS - 1) // NS

    @functools.partial(
        pl.kernel,
        out_type=jax.ShapeDtypeStruct((NC, N_NODES, D), jnp.float32),
        mesh=_vmesh(),
        scratch_types=[
            pltpu.VMEM_SHARED((N_NODES, D), jnp.float32),
            pltpu.VMEM((c_sz,), jnp.int32),
            pltpu.VMEM((c_sz,), jnp.int32),
            pltpu.VMEM((c_sz,), jnp.int32),
            pltpu.VMEM((c_sz, D), jnp.float32),
            pltpu.VMEM((c_sz, D), jnp.float32),
            pltpu.VMEM((c_sz, D), jnp.float32),
            pltpu.VMEM((c_sz, D), jnp.float32),
            pltpu.SemaphoreType.DMA,
            pltpu.SemaphoreType.DMA,
            pltpu.SemaphoreType.DMA,
        ],
    )
    def k(hw_hbm, ew_hbm, pb_hbm, src_hbm, dst_hbm, attr_hbm, out_hbm,
          agg_sh, src_v, dst_v, attr_v, h_v, e_v, p_v, msg_v,
          sem1, sem2, sem3):
        c = lax.axis_index("c")
        s = lax.axis_index("s")
        g = c * NS + s

        @pl.loop(0, c_sz)
        def _(r):
            for cc in range(D // 16):
                msg_v[r, pl.ds(cc * 16, 16)] = jnp.zeros((16,), jnp.float32)

        @pl.loop(0, n_zloop)
        def _(j):
            zc = j * NS + s

            @pl.when(zc < n_zch)
            def _():
                pltpu.sync_copy(msg_v, agg_sh.at[pl.ds(zc * z_ch, z_ch)])

        plsc.subcore_barrier()

        @pl.loop(0, n_chunk)
        def _(i):
            base = g * ept + i * c_sz
            pltpu.sync_copy(src_hbm.at[pl.ds(base, c_sz)], src_v)
            pltpu.sync_copy(dst_hbm.at[pl.ds(base, c_sz)], dst_v)
            pltpu.sync_copy(attr_hbm.at[pl.ds(base, c_sz)], attr_v)
            cp1 = pltpu.async_copy(hw_hbm.at[src_v], h_v, sem1)
            cp2 = pltpu.async_copy(ew_hbm.at[attr_v], e_v, sem2)
            cp3 = pltpu.async_copy(pb_hbm.at[attr_v], p_v, sem3)
            cp1.wait()
            cp2.wait()
            cp3.wait()

            @pl.loop(0, c_sz)
            def _(r):
                for cc in range(D // 16):
                    sl = pl.ds(cc * 16, 16)
                    m = jnp.maximum(h_v[r, sl] + e_v[r, sl], 0.0) * p_v[r, sl]
                    msg_v[r, sl] = m

            pltpu.sync_copy(msg_v, agg_sh.at[dst_v], add=True)

        plsc.subcore_barrier()

        @pl.loop(0, n_zloop)
        def _(j):
            zc = j * NS + s

            @pl.when(zc < n_zch)
            def _():
                sl = pl.ds(zc * z_ch, z_ch)
                pltpu.sync_copy(agg_sh.at[sl], out_hbm.at[c].at[sl])

    return k(hw, ew, pb, src, dst, attr)


def _tc_prep(h, rel_p, prior_p, wm):
    def body(h_ref, rel_ref, prior_ref, w_ref, hw_ref, ew_ref, pb_ref):
        w = w_ref[...]
        hw_ref[...] = jnp.dot(h_ref[...], w, preferred_element_type=jnp.float32)
        ew_ref[...] = jnp.dot(rel_ref[...], w, preferred_element_type=jnp.float32)
        pb_ref[...] = jnp.broadcast_to(prior_ref[...], (REL_PAD, D))

    return pl.pallas_call(
        body,
        out_shape=(
            jax.ShapeDtypeStruct((N_NODES, D), jnp.float32),
            jax.ShapeDtypeStruct((REL_PAD, D), jnp.float32),
            jax.ShapeDtypeStruct((REL_PAD, D), jnp.float32),
        ),
    )(h, rel_p, prior_p, wm)


def _tc_update(h, agg2, ws, wa):
    def body(h_ref, a_ref, ws_ref, wa_ref, o_ref):
        agg = a_ref[0] + a_ref[1]
        o_ref[...] = jnp.maximum(
            jnp.dot(h_ref[...], ws_ref[...], preferred_element_type=jnp.float32)
            + jnp.dot(agg, wa_ref[...], preferred_element_type=jnp.float32),
            0.0,
        )

    return pl.pallas_call(
        body,
        out_shape=jax.ShapeDtypeStruct((N_NODES, D), jnp.float32),
    )(h, agg2, ws, wa)


def kernel(x, edge_index, edge_attr, entity_emb, relation_emb, relation_prior,
           W_msg1, W_self1, W_agg1, W_msg2, W_self2, W_agg2):
    x = x.astype(jnp.int32)
    src = edge_index[0].astype(jnp.int32)
    dst = edge_index[1].astype(jnp.int32)
    attr = edge_attr.astype(jnp.int32)
    nrel = relation_emb.shape[0]
    rel_p = jnp.zeros((REL_PAD, D), jnp.float32).at[:nrel].set(relation_emb)
    prior_p = jnp.zeros((REL_PAD, 1), jnp.float32).at[:nrel].set(relation_prior)

    h = _sc_gather(entity_emb, x)
    for wm, ws, wa in ((W_msg1, W_self1, W_agg1), (W_msg2, W_self2, W_agg2)):
        hw, ew, pb = _tc_prep(h, rel_p, prior_p, wm)
        agg2 = _sc_edge(hw, ew, pb, src, dst, attr)
        h = _tc_update(h, agg2, ws, wa)
    return h
